# baseline (speedup 1.0000x reference)
import functools

import jax
import jax.numpy as jnp
from jax import lax
from jax.experimental import pallas as pl
from jax.experimental.pallas import tpu as pltpu
from jax.experimental.pallas import tpu_sc as plsc

_BATCH = 16384
_FEATURES = 64

_INFO = plsc.get_sparse_core_info()
_NC = _INFO.num_cores
_NS = _INFO.num_subcores
_NW = _NC * _NS
_BPW = _BATCH // _NW
_CH = 128
_NCH = _BPW // _CH


@functools.partial(
    pl.kernel,
    out_type=jax.ShapeDtypeStruct((_BATCH, _FEATURES), jnp.float32),
    mesh=plsc.VectorSubcoreMesh(core_axis_name="c", subcore_axis_name="s"),
    scratch_types=[
        pltpu.VMEM((_NCH, _CH), jnp.int32),
        pltpu.VMEM((_BPW, _FEATURES), jnp.float32),
        pltpu.SemaphoreType.DMA,
    ],
    compiler_params=pltpu.CompilerParams(use_tc_tiling_on_sc=False),
)
def _embed_gather(cond_hbm, table_hbm, out_hbm, idx_v, rows_v, sem):
    wid = lax.axis_index("s") * _NC + lax.axis_index("c")
    pltpu.sync_copy(cond_hbm.at[wid], idx_v)
    copies = []
    for j in range(_NCH):
        copies.append(
            pltpu.async_copy(
                table_hbm.at[idx_v.at[j]],
                rows_v.at[pl.ds(j * _CH, _CH)],
                sem,
            )
        )
    for c in copies:
        c.wait()
    pltpu.sync_copy(rows_v, out_hbm.at[pl.ds(wid * _BPW, _BPW)])


def kernel(cond, embedding_table):
    cond3 = cond.astype(jnp.int32).reshape(_NW, _NCH, _CH)
    return _embed_gather(cond3, embedding_table)


# speedup vs baseline: 2.3374x; 2.3374x over previous
import functools

import jax
import jax.numpy as jnp
from jax import lax
from jax.experimental import pallas as pl
from jax.experimental.pallas import tpu as pltpu
from jax.experimental.pallas import tpu_sc as plsc

_BATCH = 16384
_FEATURES = 64

_INFO = plsc.get_sparse_core_info()
_NC = _INFO.num_cores
_NS = _INFO.num_subcores
_NW = _NC * _NS
_BPW = _BATCH // _NW
_K = 8
_NGRP = _BPW // 16


@functools.partial(
    pl.kernel,
    out_type=jax.ShapeDtypeStruct((_FEATURES, _BATCH), jnp.float32),
    mesh=plsc.VectorSubcoreMesh(core_axis_name="c", subcore_axis_name="s"),
    scratch_types=[
        pltpu.VMEM((_BPW,), jnp.int32),
        pltpu.VMEM((_K, _FEATURES, 128), jnp.float32),
        pltpu.VMEM((_FEATURES, _BPW), jnp.float32),
        pltpu.SemaphoreType.DMA,
    ],
    compiler_params=pltpu.CompilerParams(needs_layout_passes=False),
)
def _embed_gather(cond_hbm, tabt_hbm, out_hbm, idx_v, stage_v, out_v, sem):
    wid = lax.axis_index("s") * _NC + lax.axis_index("c")
    base = wid * _BPW
    pltpu.sync_copy(cond_hbm.at[pl.ds(base, _BPW)], idx_v)

    iota = lax.iota(jnp.int32, 16)

    def group_body(g, carry):
        vec = idx_v[pl.ds(g * 16, 16)]
        offs = vec & ~127
        lanes = vec & 127
        for wave in range(16 // _K):
            copies = []
            for k in range(_K):
                j = wave * _K + k
                off = pl.multiple_of(offs[j], 128)
                copies.append(
                    pltpu.async_copy(
                        tabt_hbm.at[:, pl.ds(off, 128)], stage_v.at[k], sem
                    )
                )
            for cp in copies:
                cp.wait()
            for k in range(_K):
                j = wave * _K + k
                lane = jnp.full((16,), lanes[j], jnp.int32)
                col = jnp.full((16,), g * 16 + j, jnp.int32)
                kk = jnp.full((16,), k, jnp.int32)
                for q in range(_FEATURES // 16):
                    feats = iota + (q * 16)
                    vals = plsc.load_gather(stage_v, [kk, feats, lane])
                    plsc.store_scatter(out_v, [feats, col], vals)
        return carry

    lax.fori_loop(0, _NGRP, group_body, 0)
    pltpu.sync_copy(out_v, out_hbm.at[:, pl.ds(base, _BPW)])


def kernel(cond, embedding_table):
    out_t = _embed_gather(cond.astype(jnp.int32), embedding_table.T)
    return out_t.T


# speedup vs baseline: 3.8094x; 1.6298x over previous
import functools

import jax
import jax.numpy as jnp
from jax import lax
from jax.experimental import pallas as pl
from jax.experimental.pallas import tpu as pltpu
from jax.experimental.pallas import tpu_sc as plsc

_BATCH = 16384
_FEATURES = 64
_CLASSES = 1000000

_INFO = plsc.get_sparse_core_info()
_NC = _INFO.num_cores
_NS = _INFO.num_subcores
_NW = _NC * _NS
_COLS = (_CLASSES + 127) // 128
_RPW = (_COLS + _NW - 1) // _NW
_CPC = 4
_NCHUNK = (_RPW + _CPC - 1) // _CPC
_NPAIR = (_NCHUNK + 1) // 2
_MAXOFF = (_COLS - _CPC) * 128
_NGRP = _BATCH // 16
_RING = 8
_SENT = 0x7FFF0000


@functools.partial(
    pl.kernel,
    out_type=jax.ShapeDtypeStruct((_BATCH * _FEATURES,), jnp.float32),
    mesh=plsc.VectorSubcoreMesh(core_axis_name="c", subcore_axis_name="s"),
    scratch_types=[
        pltpu.VMEM((_BATCH,), jnp.int32),
        pltpu.VMEM((_BATCH,), jnp.int32),
        pltpu.VMEM((_BATCH,), jnp.int32),
        pltpu.VMEM((2, _FEATURES, _CPC * 128), jnp.float32),
        pltpu.VMEM((_RING, _FEATURES), jnp.float32),
        pltpu.SemaphoreType.DMA,
        pltpu.SemaphoreType.DMA,
        pltpu.SemaphoreType.DMA,
    ],
    compiler_params=pltpu.CompilerParams(needs_layout_passes=False),
)
def _embed_gather(cond_hbm, tabt_hbm, out_hbm, idx_all, match_v, match_p,
                  stage, rowbufs, semb0, semb1, semr):
    wid = lax.axis_index("s") * _NC + lax.axis_index("c")
    lo = wid * _RPW
    hi = jnp.minimum(lo + _RPW, _COLS)
    iota = lax.iota(jnp.int32, 16)
    sems = (semb0, semb1)

    def fire(b, c):
        off = pl.multiple_of(
            jnp.minimum(lo + c * _CPC, _COLS - _CPC) * 128, 128)
        return pltpu.async_copy(
            tabt_hbm.at[:, pl.ds(off, _CPC * 128)], stage.at[b], sems[b])

    def drain(b):
        pltpu.make_async_copy(
            tabt_hbm.at[:, pl.ds(0, _CPC * 128)], stage.at[b], sems[b]).wait()

    fire(0, 0)

    pltpu.sync_copy(cond_hbm.at[pl.ds(0, _BATCH)], idx_all)

    def prefill(j, carry):
        match_v[pl.ds(j * 16, 16)] = jnp.full((16,), _SENT, jnp.int32)
        return carry

    lax.fori_loop(0, _NGRP, prefill, 0)

    def scan_body(g, cnt):
        vec = idx_all[pl.ds(g * 16, 16)]
        cols = vec >> 7
        msk = (cols >= lo) & (cols < hi)
        plsc.store_compressed(match_v.at[pl.ds(cnt, 16)], vec, mask=msk)
        plsc.store_compressed(match_p.at[pl.ds(cnt, 16)],
                              iota + g * 16, mask=msk)
        return cnt + plsc.all_reduce_population_count(msk)[0]

    nmatch = lax.fori_loop(0, _NGRP, scan_body, 0)
    ngrp_m = (nmatch + 15) >> 4

    def process(c, b, fired0):
        c0 = lo + c * _CPC
        cs = jnp.minimum(c0, _COLS - _CPC)
        base_lane = cs * 128

        def grp_body(j, fired):
            mv = match_v[pl.ds(j * 16, 16)]
            cols = mv >> 7
            m0 = (cols >= c0) & (cols < c0 + _CPC)

            def w_cond(state):
                m, _ = state
                return plsc.all_reduce_population_count(m)[0] > 0

            def w_body(state):
                m, fired = state
                k = plsc.all_reduce_ffs(m)[0]
                kk = jnp.full((16,), j * 16 + k, jnp.int32)
                vv = plsc.load_gather(match_v, [kk])[0]
                pp = plsc.load_gather(match_p, [kk])[0]
                lane = jnp.full((16,), vv - base_lane, jnp.int32)
                slot = fired & (_RING - 1)

                @pl.when(fired >= _RING)
                def _():
                    pltpu.make_async_copy(
                        rowbufs.at[0],
                        out_hbm.at[pl.ds(0, _FEATURES)], semr).wait()

                for q in range(_FEATURES // 16):
                    vals = plsc.load_gather(
                        stage, [jnp.full((16,), b, jnp.int32),
                                iota + q * 16, lane])
                    rowbufs[slot, pl.ds(q * 16, 16)] = vals
                pltpu.async_copy(
                    rowbufs.at[slot],
                    out_hbm.at[pl.ds(pp * _FEATURES, _FEATURES)], semr)
                return m & (iota != k), fired + 1

            m_fin, fired = lax.while_loop(w_cond, w_body, (m0, fired))
            return fired

        return lax.fori_loop(0, ngrp_m, grp_body, fired0)

    def pair_body(i, fired):
        fire(1, 2 * i + 1)
        drain(0)
        fired = process(2 * i, 0, fired)
        fire(0, 2 * i + 2)
        drain(1)
        fired = process(2 * i + 1, 1, fired)
        return fired

    fired = lax.fori_loop(0, _NPAIR, pair_body, 0)
    drain(0)

    def rdrain(j, carry):
        pltpu.make_async_copy(
            rowbufs.at[0], out_hbm.at[pl.ds(0, _FEATURES)], semr).wait()
        return carry

    lax.fori_loop(0, jnp.minimum(fired, _RING), rdrain, 0)


def kernel(cond, embedding_table):
    out_flat = _embed_gather(cond.astype(jnp.int32), embedding_table.T)
    return out_flat.reshape(_BATCH, _FEATURES)


# speedup vs baseline: 3.8315x; 1.0058x over previous
import functools

import jax
import jax.numpy as jnp
from jax import lax
from jax.experimental import pallas as pl
from jax.experimental.pallas import tpu as pltpu
from jax.experimental.pallas import tpu_sc as plsc

_BATCH = 16384
_FEATURES = 64
_CLASSES = 1000000

_INFO = plsc.get_sparse_core_info()
_NC = _INFO.num_cores
_NS = _INFO.num_subcores
_NW = _NC * _NS
_COLS = (_CLASSES + 127) // 128
_RPW = (_COLS + _NW - 1) // _NW
_CPC = 4
_NCHUNK = (_RPW + _CPC - 1) // _CPC
_NPAIR = (_NCHUNK + 1) // 2
_MAXOFF = (_COLS - _CPC) * 128
_NGRP = _BATCH // 16
_RING = 8
_SENT = 0x7FFF0000


@functools.partial(
    pl.kernel,
    out_type=jax.ShapeDtypeStruct((_BATCH * _FEATURES,), jnp.float32),
    mesh=plsc.VectorSubcoreMesh(core_axis_name="c", subcore_axis_name="s"),
    scratch_types=[
        pltpu.VMEM((_BATCH,), jnp.int32),
        pltpu.VMEM((_BATCH,), jnp.int32),
        pltpu.VMEM((_BATCH,), jnp.int32),
        pltpu.VMEM((2, _FEATURES, _CPC * 128), jnp.float32),
        pltpu.VMEM((_RING, _FEATURES), jnp.float32),
        pltpu.SemaphoreType.DMA,
        pltpu.SemaphoreType.DMA,
        pltpu.SemaphoreType.DMA,
    ],
    compiler_params=pltpu.CompilerParams(needs_layout_passes=False),
)
def _embed_gather(cond_hbm, tabt_hbm, out_hbm, idx_all, match_v, match_p,
                  stage, rowbufs, semb0, semb1, semr):
    wid = lax.axis_index("s") * _NC + lax.axis_index("c")
    lo = wid * _RPW
    hi = jnp.minimum(lo + _RPW, _COLS)
    iota = lax.iota(jnp.int32, 16)
    sems = (semb0, semb1)

    def fire(b, c):
        off = pl.multiple_of(
            jnp.minimum(lo + c * _CPC, _COLS - _CPC) * 128, 128)
        return pltpu.async_copy(
            tabt_hbm.at[:, pl.ds(off, _CPC * 128)], stage.at[b], sems[b])

    def drain(b):
        pltpu.make_async_copy(
            tabt_hbm.at[:, pl.ds(0, _CPC * 128)], stage.at[b], sems[b]).wait()

    fire(0, 0)
    fire(1, 1)

    pltpu.sync_copy(cond_hbm.at[pl.ds(0, _BATCH)], idx_all)

    def scan_body(g, cnt):
        vec = idx_all[pl.ds(g * 16, 16)]
        cols = vec >> 7
        msk = (cols >= lo) & (cols < hi)
        plsc.store_compressed(match_v.at[pl.ds(cnt, 16)], vec, mask=msk)
        plsc.store_compressed(match_p.at[pl.ds(cnt, 16)],
                              iota + g * 16, mask=msk)
        return cnt + plsc.all_reduce_population_count(msk)[0]

    nmatch = lax.fori_loop(0, _NGRP, scan_body, 0)
    ngrp_m = (nmatch + 15) >> 4

    tail_base = (nmatch >> 4) << 4
    tgrp = match_v[pl.ds(tail_base, 16)]
    match_v[pl.ds(tail_base, 16)] = jnp.where(
        iota >= (nmatch & 15), jnp.full((16,), _SENT, jnp.int32), tgrp)

    def process(c, b, fired0):
        c0 = lo + c * _CPC
        cs = jnp.minimum(c0, _COLS - _CPC)
        base_lane = cs * 128

        def grp_body(j, fired):
            mv = match_v[pl.ds(j * 16, 16)]
            cols = mv >> 7
            m0 = (cols >= c0) & (cols < c0 + _CPC)

            def w_cond(state):
                m, _ = state
                return plsc.all_reduce_population_count(m)[0] > 0

            def w_body(state):
                m, fired = state
                k = plsc.all_reduce_ffs(m)[0]
                kk = jnp.full((16,), j * 16 + k, jnp.int32)
                vv = plsc.load_gather(match_v, [kk])[0]
                pp = plsc.load_gather(match_p, [kk])[0]
                lane = jnp.full((16,), vv - base_lane, jnp.int32)
                slot = fired & (_RING - 1)

                @pl.when(fired >= _RING)
                def _():
                    pltpu.make_async_copy(
                        rowbufs.at[0],
                        out_hbm.at[pl.ds(0, _FEATURES)], semr).wait()

                for q in range(_FEATURES // 16):
                    vals = plsc.load_gather(
                        stage, [jnp.full((16,), b, jnp.int32),
                                iota + q * 16, lane])
                    rowbufs[slot, pl.ds(q * 16, 16)] = vals
                pltpu.async_copy(
                    rowbufs.at[slot],
                    out_hbm.at[pl.ds(pp * _FEATURES, _FEATURES)], semr)
                return m & (iota != k), fired + 1

            m_fin, fired = lax.while_loop(w_cond, w_body, (m0, fired))
            return fired

        return lax.fori_loop(0, ngrp_m, grp_body, fired0)

    def pair_body(i, fired):
        drain(0)
        fired = process(2 * i, 0, fired)
        fire(0, 2 * i + 2)
        drain(1)
        fired = process(2 * i + 1, 1, fired)
        fire(1, 2 * i + 3)
        return fired

    fired = lax.fori_loop(0, _NPAIR, pair_body, 0)
    drain(0)
    drain(1)

    def rdrain(j, carry):
        pltpu.make_async_copy(
            rowbufs.at[0], out_hbm.at[pl.ds(0, _FEATURES)], semr).wait()
        return carry

    lax.fori_loop(0, jnp.minimum(fired, _RING), rdrain, 0)


def kernel(cond, embedding_table):
    out_flat = _embed_gather(cond.astype(jnp.int32), embedding_table.T)
    return out_flat.reshape(_BATCH, _FEATURES)


# speedup vs baseline: 4.0014x; 1.0443x over previous
import functools

import jax
import jax.numpy as jnp
from jax import lax
from jax.experimental import pallas as pl
from jax.experimental.pallas import tpu as pltpu
from jax.experimental.pallas import tpu_sc as plsc

_BATCH = 16384
_FEATURES = 64
_CLASSES = 1000000

_INFO = plsc.get_sparse_core_info()
_NC = _INFO.num_cores
_NS = _INFO.num_subcores
_NW = _NC * _NS
_COLS = (_CLASSES + 127) // 128
_RPW = (_COLS + _NW - 1) // _NW
_CPC = 4
_NCHUNK = (_RPW + _CPC - 1) // _CPC
_NPAIR = (_NCHUNK + 1) // 2
_MAXOFF = (_COLS - _CPC) * 128
_NGRP = _BATCH // 16
_RING = 8
_SENT = 0x7FFF0000


@functools.partial(
    pl.kernel,
    out_type=jax.ShapeDtypeStruct((_BATCH, _FEATURES), jnp.float32),
    mesh=plsc.VectorSubcoreMesh(core_axis_name="c", subcore_axis_name="s"),
    scratch_types=[
        pltpu.VMEM((_BATCH,), jnp.int32),
        pltpu.VMEM((_BATCH,), jnp.int32),
        pltpu.VMEM((_BATCH,), jnp.int32),
        pltpu.VMEM((2, _FEATURES, _CPC * 128), jnp.float32),
        pltpu.VMEM((_RING, 1, _FEATURES), jnp.float32),
        pltpu.SemaphoreType.DMA,
        pltpu.SemaphoreType.DMA,
        pltpu.SemaphoreType.DMA,
    ],
    compiler_params=pltpu.CompilerParams(needs_layout_passes=False),
)
def _embed_gather(cond_hbm, tabt_hbm, out_hbm, idx_all, match_v, match_p,
                  stage, rowbufs, semb0, semb1, semr):
    wid = lax.axis_index("s") * _NC + lax.axis_index("c")
    lo = wid * _RPW
    hi = jnp.minimum(lo + _RPW, _COLS)
    iota = lax.iota(jnp.int32, 16)
    sems = (semb0, semb1)

    def fire(b, c):
        off = pl.multiple_of(
            jnp.minimum(lo + c * _CPC, _COLS - _CPC) * 128, 128)
        return pltpu.async_copy(
            tabt_hbm.at[:, pl.ds(off, _CPC * 128)], stage.at[b], sems[b])

    def drain(b):
        pltpu.make_async_copy(
            tabt_hbm.at[:, pl.ds(0, _CPC * 128)], stage.at[b], sems[b]).wait()

    fire(0, 0)
    fire(1, 1)

    pltpu.sync_copy(cond_hbm.at[pl.ds(0, _BATCH)], idx_all)

    def scan_body(g, cnt):
        vec = idx_all[pl.ds(g * 16, 16)]
        cols = vec >> 7
        msk = (cols >= lo) & (cols < hi)
        plsc.store_compressed(match_v.at[pl.ds(cnt, 16)], vec, mask=msk)
        plsc.store_compressed(match_p.at[pl.ds(cnt, 16)],
                              iota + g * 16, mask=msk)
        return cnt + plsc.all_reduce_population_count(msk)[0]

    nmatch = lax.fori_loop(0, _NGRP, scan_body, 0)
    ngrp_m = (nmatch + 15) >> 4

    tail_base = (nmatch >> 4) << 4
    tgrp = match_v[pl.ds(tail_base, 16)]
    match_v[pl.ds(tail_base, 16)] = jnp.where(
        iota >= (nmatch & 15), jnp.full((16,), _SENT, jnp.int32), tgrp)

    def process(c, b, fired0):
        c0 = lo + c * _CPC
        cs = jnp.minimum(c0, _COLS - _CPC)
        base_lane = cs * 128

        def grp_body(j, fired):
            mv = match_v[pl.ds(j * 16, 16)]
            cols = mv >> 7
            m0 = (cols >= c0) & (cols < c0 + _CPC)

            def w_cond(state):
                m, _ = state
                return plsc.all_reduce_population_count(m)[0] > 0

            def w_body(state):
                m, fired = state
                k = plsc.all_reduce_ffs(m)[0]
                kk = jnp.full((16,), j * 16 + k, jnp.int32)
                vv = plsc.load_gather(match_v, [kk])[0]
                pp = plsc.load_gather(match_p, [kk])[0]
                lane = jnp.full((16,), vv - base_lane, jnp.int32)
                slot = fired & (_RING - 1)

                @pl.when(fired >= _RING)
                def _():
                    pltpu.make_async_copy(
                        rowbufs.at[0],
                        out_hbm.at[pl.ds(0, 1), :], semr).wait()

                for q in range(_FEATURES // 16):
                    vals = plsc.load_gather(
                        stage, [jnp.full((16,), b, jnp.int32),
                                iota + q * 16, lane])
                    rowbufs[slot, 0, pl.ds(q * 16, 16)] = vals
                pltpu.async_copy(
                    rowbufs.at[slot],
                    out_hbm.at[pl.ds(pp, 1), :], semr)
                return m & (iota != k), fired + 1

            m_fin, fired = lax.while_loop(w_cond, w_body, (m0, fired))
            return fired

        return lax.fori_loop(0, ngrp_m, grp_body, fired0)

    def pair_body(i, fired):
        drain(0)
        fired = process(2 * i, 0, fired)
        fire(0, 2 * i + 2)
        drain(1)
        fired = process(2 * i + 1, 1, fired)
        fire(1, 2 * i + 3)
        return fired

    fired = lax.fori_loop(0, _NPAIR, pair_body, 0)
    drain(0)
    drain(1)

    def rdrain(j, carry):
        pltpu.make_async_copy(
            rowbufs.at[0], out_hbm.at[pl.ds(0, 1), :], semr).wait()
        return carry

    lax.fori_loop(0, jnp.minimum(fired, _RING), rdrain, 0)


def kernel(cond, embedding_table):
    return _embed_gather(cond.astype(jnp.int32), embedding_table.T)


# speedup vs baseline: 4.0586x; 1.0143x over previous
import functools

import jax
import jax.numpy as jnp
from jax import lax
from jax.experimental import pallas as pl
from jax.experimental.pallas import tpu as pltpu
from jax.experimental.pallas import tpu_sc as plsc

_BATCH = 16384
_FEATURES = 64
_CLASSES = 1000000

_INFO = plsc.get_sparse_core_info()
_NC = _INFO.num_cores
_NS = _INFO.num_subcores
_NW = _NC * _NS
_COLS = (_CLASSES + 127) // 128
_RPW = (_COLS + _NW - 1) // _NW
_CPC = 4
_NCHUNK = (_RPW + _CPC - 1) // _CPC
_NPAIR = (_NCHUNK + 1) // 2
_MAXOFF = (_COLS - _CPC) * 128
_NGRP = _BATCH // 16
_RING = 8
_SENT = 0x7FFF0000


@functools.partial(
    pl.kernel,
    out_type=jax.ShapeDtypeStruct((_BATCH, _FEATURES), jnp.float32),
    mesh=plsc.VectorSubcoreMesh(core_axis_name="c", subcore_axis_name="s"),
    scratch_types=[
        pltpu.VMEM((_BATCH,), jnp.int32),
        pltpu.VMEM((_BATCH,), jnp.int32),
        pltpu.VMEM((_BATCH,), jnp.int32),
        pltpu.VMEM((256,), jnp.int32),
        pltpu.VMEM((2, _FEATURES, _CPC * 128), jnp.float32),
        pltpu.VMEM((_RING, 1, _FEATURES), jnp.float32),
        pltpu.SemaphoreType.DMA,
        pltpu.SemaphoreType.DMA,
        pltpu.SemaphoreType.DMA,
    ],
    compiler_params=pltpu.CompilerParams(needs_layout_passes=False),
)
def _embed_gather(cond_hbm, tabt_hbm, out_hbm, idx_all, match_v, match_p,
                  colmap, stage, rowbufs, semb0, semb1, semr):
    wid = lax.axis_index("s") * _NC + lax.axis_index("c")
    lo = wid * _RPW
    hi = jnp.minimum(lo + _RPW, _COLS)
    iota = lax.iota(jnp.int32, 16)
    sems = (semb0, semb1)
    one16 = jnp.full((16,), 1, jnp.int32)

    def fire_col(b, k, off):
        pltpu.async_copy(
            tabt_hbm.at[:, pl.ds(pl.multiple_of(off, 128), 128)],
            stage.at[b, :, pl.ds(k * 128, 128)], sems[b])

    def fire_chunk_uncond(b, c):
        for k in range(_CPC):
            fire_col(b, k, (lo + c * _CPC + k) * 128)
        return _CPC

    def fire_chunk(b, c):
        nf = 0
        for k in range(_CPC):
            flag = plsc.load_gather(
                colmap, [jnp.full((16,), c * _CPC + k, jnp.int32)])[0]

            @pl.when(flag > 0)
            def _():
                fire_col(b, k, (lo + c * _CPC + k) * 128)

            nf = nf + flag
        return nf

    def drain_n(b, n):
        def dbody(j, carry):
            pltpu.make_async_copy(
                tabt_hbm.at[:, pl.ds(0, 128)],
                stage.at[b, :, pl.ds(0, 128)], sems[b]).wait()
            return carry

        lax.fori_loop(0, n, dbody, 0)

    f0 = fire_chunk_uncond(0, 0)
    f1 = fire_chunk_uncond(1, 1)

    pltpu.sync_copy(cond_hbm.at[pl.ds(0, _BATCH)], idx_all)

    def cbody(j, carry):
        colmap[pl.ds(j * 16, 16)] = jnp.full((16,), 0, jnp.int32)
        return carry

    lax.fori_loop(0, 16, cbody, 0)

    def scan_body(g, cnt):
        vec = idx_all[pl.ds(g * 16, 16)]
        cols = vec >> 7
        msk = (cols >= lo) & (cols < hi)
        plsc.store_compressed(match_v.at[pl.ds(cnt, 16)], vec, mask=msk)
        plsc.store_compressed(match_p.at[pl.ds(cnt, 16)],
                              iota + g * 16, mask=msk)
        plsc.store_scatter(colmap, [cols - lo], one16, mask=msk)
        return cnt + plsc.all_reduce_population_count(msk)[0]

    nmatch = lax.fori_loop(0, _NGRP, scan_body, 0)
    ngrp_m = (nmatch + 15) >> 4

    tail_base = (nmatch >> 4) << 4
    tgrp = match_v[pl.ds(tail_base, 16)]
    match_v[pl.ds(tail_base, 16)] = jnp.where(
        iota >= (nmatch & 15), jnp.full((16,), _SENT, jnp.int32), tgrp)

    def process(c, b, fired0):
        c0 = lo + c * _CPC
        base_lane = c0 * 128

        def grp_body(j, fired):
            mv = match_v[pl.ds(j * 16, 16)]
            cols = mv >> 7
            m0 = (cols >= c0) & (cols < c0 + _CPC)

            def w_cond(state):
                m, _ = state
                return plsc.all_reduce_population_count(m)[0] > 0

            def w_body(state):
                m, fired = state
                k = plsc.all_reduce_ffs(m)[0]
                kk = jnp.full((16,), j * 16 + k, jnp.int32)
                vv = plsc.load_gather(match_v, [kk])[0]
                pp = plsc.load_gather(match_p, [kk])[0]
                lane = jnp.full((16,), vv - base_lane, jnp.int32)
                slot = fired & (_RING - 1)

                @pl.when(fired >= _RING)
                def _():
                    pltpu.make_async_copy(
                        rowbufs.at[0],
                        out_hbm.at[pl.ds(0, 1), :], semr).wait()

                for q in range(_FEATURES // 16):
                    vals = plsc.load_gather(
                        stage, [jnp.full((16,), b, jnp.int32),
                                iota + q * 16, lane])
                    rowbufs[slot, 0, pl.ds(q * 16, 16)] = vals
                pltpu.async_copy(
                    rowbufs.at[slot],
                    out_hbm.at[pl.ds(pp, 1), :], semr)
                return m & (iota != k), fired + 1

            m_fin, fired = lax.while_loop(w_cond, w_body, (m0, fired))
            return fired

        return lax.fori_loop(0, ngrp_m, grp_body, fired0)

    def pair_body(i, state):
        fired, f0, f1 = state
        drain_n(0, f0)
        fired = process(2 * i, 0, fired)
        f0 = fire_chunk(0, 2 * i + 2)
        drain_n(1, f1)
        fired = process(2 * i + 1, 1, fired)
        f1 = fire_chunk(1, 2 * i + 3)
        return fired, f0, f1

    fired, f0, f1 = lax.fori_loop(0, _NPAIR, pair_body, (0, f0, f1))
    drain_n(0, f0)
    drain_n(1, f1)

    def rdrain(j, carry):
        pltpu.make_async_copy(
            rowbufs.at[0], out_hbm.at[pl.ds(0, 1), :], semr).wait()
        return carry

    lax.fori_loop(0, jnp.minimum(fired, _RING), rdrain, 0)


def kernel(cond, embedding_table):
    return _embed_gather(cond.astype(jnp.int32), embedding_table.T)


# speedup vs baseline: 4.1066x; 1.0118x over previous
import functools

import jax
import jax.numpy as jnp
from jax import lax
from jax.experimental import pallas as pl
from jax.experimental.pallas import tpu as pltpu
from jax.experimental.pallas import tpu_sc as plsc

_BATCH = 16384
_FEATURES = 64
_CLASSES = 1000000

_INFO = plsc.get_sparse_core_info()
_NC = _INFO.num_cores
_NS = _INFO.num_subcores
_NW = _NC * _NS
_COLS = (_CLASSES + 127) // 128
_RPW = (_COLS + _NW - 1) // _NW
_CPC = 4
_NCHUNK = (_RPW + _CPC - 1) // _CPC
_NPAIR = (_NCHUNK + 1) // 2
_MAXOFF = (_COLS - _CPC) * 128
_NGRP = _BATCH // 16
_RING = 8
_SENT = 0x7FFF0000


@functools.partial(
    pl.kernel,
    out_type=jax.ShapeDtypeStruct((_BATCH, _FEATURES), jnp.float32),
    mesh=plsc.VectorSubcoreMesh(core_axis_name="c", subcore_axis_name="s"),
    scratch_types=[
        pltpu.VMEM((_BATCH,), jnp.int32),
        pltpu.VMEM((_BATCH,), jnp.int32),
        pltpu.VMEM((_BATCH,), jnp.int32),
        pltpu.VMEM((256,), jnp.int32),
        pltpu.VMEM((2, _FEATURES, _CPC * 128), jnp.float32),
        pltpu.VMEM((_RING, 1, _FEATURES), jnp.float32),
        pltpu.SemaphoreType.DMA,
        pltpu.SemaphoreType.DMA,
        pltpu.SemaphoreType.DMA,
    ],
    compiler_params=pltpu.CompilerParams(needs_layout_passes=False),
)
def _embed_gather(cond_hbm, tabt_hbm, out_hbm, idx_all, match_v, match_p,
                  colmap, stage, rowbufs, semb0, semb1, semr):
    wid = lax.axis_index("s") * _NC + lax.axis_index("c")
    lo = wid * _RPW
    hi = jnp.minimum(lo + _RPW, _COLS)
    iota = lax.iota(jnp.int32, 16)
    sems = (semb0, semb1)
    one16 = jnp.full((16,), 1, jnp.int32)

    def fire_col(b, k, off):
        pltpu.async_copy(
            tabt_hbm.at[:, pl.ds(pl.multiple_of(off, 128), 128)],
            stage.at[b, :, pl.ds(k * 128, 128)], sems[b])

    def fire_chunk_uncond(b, c):
        for k in range(_CPC):
            fire_col(b, k, (lo + c * _CPC + k) * 128)
        return _CPC

    def fire_chunk(b, c):
        nf = 0
        for k in range(_CPC):
            flag = plsc.load_gather(
                colmap, [jnp.full((16,), c * _CPC + k, jnp.int32)])[0]

            @pl.when(flag > 0)
            def _():
                fire_col(b, k, (lo + c * _CPC + k) * 128)

            nf = nf + flag
        return nf

    def drain_n(b, n):
        def dbody(j, carry):
            pltpu.make_async_copy(
                tabt_hbm.at[:, pl.ds(0, 128)],
                stage.at[b, :, pl.ds(0, 128)], sems[b]).wait()
            return carry

        lax.fori_loop(0, n, dbody, 0)

    f0 = fire_chunk_uncond(0, 0)
    f1 = fire_chunk_uncond(1, 1)

    pltpu.sync_copy(cond_hbm.at[pl.ds(0, _BATCH)], idx_all)

    def cbody(j, carry):
        colmap[pl.ds(j * 16, 16)] = jnp.full((16,), 0, jnp.int32)
        return carry

    lax.fori_loop(0, 16, cbody, 0)

    def scan_body(g2, cnt):
        for u in range(2):
            g = g2 * 2 + u
            vec = idx_all[pl.ds(g * 16, 16)]
            cols = vec >> 7
            msk = (cols >= lo) & (cols < hi)
            plsc.store_compressed(match_v.at[pl.ds(cnt, 16)], vec, mask=msk)
            plsc.store_compressed(match_p.at[pl.ds(cnt, 16)],
                                  iota + g * 16, mask=msk)
            plsc.store_scatter(colmap, [cols - lo], one16, mask=msk)
            cnt = cnt + plsc.all_reduce_population_count(msk)[0]
        return cnt

    nmatch = lax.fori_loop(0, _NGRP // 2, scan_body, 0)
    ngrp_m = (nmatch + 15) >> 4

    tail_base = (nmatch >> 4) << 4
    tgrp = match_v[pl.ds(tail_base, 16)]
    match_v[pl.ds(tail_base, 16)] = jnp.where(
        iota >= (nmatch & 15), jnp.full((16,), _SENT, jnp.int32), tgrp)

    def process(c, b, fired0):
        c0 = lo + c * _CPC
        base_lane = c0 * 128

        def grp_body(j, fired):
            mv = match_v[pl.ds(j * 16, 16)]
            cols = mv >> 7
            m0 = (cols >= c0) & (cols < c0 + _CPC)

            def w_cond(state):
                m, _ = state
                return plsc.all_reduce_population_count(m)[0] > 0

            def w_body(state):
                m, fired = state
                k = plsc.all_reduce_ffs(m)[0]
                kk = jnp.full((16,), j * 16 + k, jnp.int32)
                vv = plsc.load_gather(match_v, [kk])[0]
                pp = plsc.load_gather(match_p, [kk])[0]
                lane = jnp.full((16,), vv - base_lane, jnp.int32)
                slot = fired & (_RING - 1)

                @pl.when(fired >= _RING)
                def _():
                    pltpu.make_async_copy(
                        rowbufs.at[0],
                        out_hbm.at[pl.ds(0, 1), :], semr).wait()

                for q in range(_FEATURES // 16):
                    vals = plsc.load_gather(
                        stage, [jnp.full((16,), b, jnp.int32),
                                iota + q * 16, lane])
                    rowbufs[slot, 0, pl.ds(q * 16, 16)] = vals
                pltpu.async_copy(
                    rowbufs.at[slot],
                    out_hbm.at[pl.ds(pp, 1), :], semr)
                return m & (iota != k), fired + 1

            m_fin, fired = lax.while_loop(w_cond, w_body, (m0, fired))
            return fired

        return lax.fori_loop(0, ngrp_m, grp_body, fired0)

    def pair_body(i, state):
        fired, f0, f1 = state
        drain_n(0, f0)
        fired = process(2 * i, 0, fired)
        f0 = fire_chunk(0, 2 * i + 2)
        drain_n(1, f1)
        fired = process(2 * i + 1, 1, fired)
        f1 = fire_chunk(1, 2 * i + 3)
        return fired, f0, f1

    fired, f0, f1 = lax.fori_loop(0, _NPAIR, pair_body, (0, f0, f1))
    drain_n(0, f0)
    drain_n(1, f1)

    def rdrain(j, carry):
        pltpu.make_async_copy(
            rowbufs.at[0], out_hbm.at[pl.ds(0, 1), :], semr).wait()
        return carry

    lax.fori_loop(0, jnp.minimum(fired, _RING), rdrain, 0)


def kernel(cond, embedding_table):
    return _embed_gather(cond.astype(jnp.int32), embedding_table.T)
